# baseline (device time: 79266 ns/iter reference)
import jax
import jax.numpy as jnp
from jax import lax
from jax.experimental import pallas as pl
from jax.experimental.pallas import tpu as pltpu

N_DEV = 16
SQ = 1024
SKV = 1024
H_PER = 8
DH = 128
D_MODEL = 1024
HALF = SQ // 2
CHALF = 512
SCALE = 0.08838834764831843

SCHED_A = ((3, 2), (1, 1), (4, 4), (8, 8))
SCHED_B = ((4, 4), (8, 8), (3, 2), (1, 1))
RS_HALF = (512, 256, 128, 64)
RS_OFF = (0, 512, 768, 896)
AG_SIZE = (64, 128, 256, 512)


def kernel(x, Wq, K_ext, V_ext, Wo):
    d = lax.axis_index("i")

    x2 = x.reshape(SQ, 1024).astype(jnp.bfloat16)
    Wq_i = lax.dynamic_slice(Wq, (0, d * D_MODEL), (1024, D_MODEL)).astype(
        jnp.bfloat16
    )
    Wo_i = lax.dynamic_slice(Wo, (d * D_MODEL, 0), (D_MODEL, 1024)).astype(
        jnp.bfloat16
    )
    K2 = K_ext.reshape(SKV, H_PER * DH).astype(jnp.bfloat16)
    V2 = V_ext.reshape(SKV, H_PER * DH).astype(jnp.bfloat16)

    def body(x_ref, wq_ref, k_ref, v_ref, wo_ref, out_ref, g_ref,
             stage_ref, rsbuf_ref, q_scr, ctx_scr, bias_scr,
             rs_send, rs_recv, ag_send, ag_recv):
        my = lax.axis_index("i")
        COLS = (pl.ds(0, CHALF), pl.ds(CHALF, CHALF))

        q_scr[...] = (
            jnp.dot(x_ref[...], wq_ref[...],
                    preferred_element_type=jnp.float32) * SCALE
        ).astype(jnp.bfloat16)

        qi = lax.broadcasted_iota(jnp.int32, (SQ, SKV), 0)
        ki = lax.broadcasted_iota(jnp.int32, (SQ, SKV), 1)
        mask = (jnp.abs(qi - ki) <= 128) | (ki < 32) | (qi < 32)
        bias_scr[...] = jnp.where(mask, 0.0, -1e9)

        barrier_sem = pltpu.get_barrier_semaphore()
        for m_, _ in SCHED_A:
            pl.semaphore_signal(
                barrier_sem, inc=1,
                device_id=(my ^ m_,), device_id_type=pl.DeviceIdType.MESH,
            )
        pl.semaphore_wait(barrier_sem, 4)

        def compute_half(ro):
            def head_body(h, carry):
                col = h * DH
                q_h = q_scr[pl.ds(ro, HALF), pl.ds(col, DH)]
                k_h = k_ref[:, pl.ds(col, DH)]
                s = lax.dot_general(
                    q_h, k_h, (((1,), (1,)), ((), ())),
                    preferred_element_type=jnp.float32,
                ) + bias_scr[pl.ds(ro, HALF), :]
                w = jnp.exp(s)
                recip = 1.0 / jnp.sum(w, axis=1, keepdims=True)
                ctx = jnp.dot(
                    w.astype(jnp.bfloat16), v_ref[:, pl.ds(col, DH)],
                    preferred_element_type=jnp.float32,
                )
                ctx_scr[pl.ds(ro, HALF), pl.ds(col, DH)] = (
                    ctx * recip
                ).astype(jnp.bfloat16)
                return carry

            lax.fori_loop(0, H_PER, head_body, 0)
            out_ref[pl.ds(ro, HALF), :] = jnp.dot(
                ctx_scr[pl.ds(ro, HALF), :], wo_ref[...],
                preferred_element_type=jnp.float32,
            )

        def rs_parts(k, lo, sched, side, sem_idx):
            m_, key = sched[k]
            half = RS_HALF[k]
            off = RS_OFF[k]
            hb = (my & key) != 0
            keep_lo = lo + jnp.where(hb, half, 0)
            send_lo = lo + jnp.where(hb, 0, half)
            cs = COLS[side]

            def stage():
                stage_ref[pl.ds(off, half), cs] = (
                    out_ref[pl.ds(send_lo, half), cs].astype(jnp.bfloat16)
                )

            rdma = pltpu.make_async_remote_copy(
                src_ref=stage_ref.at[pl.ds(off, half), cs],
                dst_ref=rsbuf_ref.at[pl.ds(off, half), cs],
                send_sem=rs_send.at[sem_idx],
                recv_sem=rs_recv.at[sem_idx],
                device_id=(my ^ m_,),
                device_id_type=pl.DeviceIdType.MESH,
            )

            def acc():
                rows = pl.ds(keep_lo, half)
                out_ref[rows, cs] = (
                    out_ref[rows, cs]
                    + rsbuf_ref[pl.ds(off, half), cs].astype(jnp.float32)
                )

            return stage, rdma, acc, keep_lo

        bitA = (my & SCHED_A[0][1]) != 0
        bitB = (my & SCHED_B[0][1]) != 0
        sendA = jnp.where(bitA, 0, HALF).astype(jnp.int32)
        same = bitA == bitB

        stA, rdma_a0, accA, lo_a = rs_parts(0, jnp.int32(0), SCHED_A, 0, 0)
        stB, rdma_b0, accB, lo_b = rs_parts(0, jnp.int32(0), SCHED_B, 1, 4)

        compute_half(sendA)
        stA()
        rdma_a0.start()

        @pl.when(same)
        def _():
            stB()
            rdma_b0.start()

        compute_half(HALF - sendA)

        @pl.when(jnp.logical_not(same))
        def _():
            stB()
            rdma_b0.start()

        rdma_a0.wait()
        accA()
        rdma_b0.wait()
        accB()

        for k in (1, 2, 3):
            stA, ra, accA, lo_a = rs_parts(k, lo_a, SCHED_A, 0, k)
            stB, rb, accB, lo_b = rs_parts(k, lo_b, SCHED_B, 1, 4 + k)
            stA()
            ra.start()
            stB()
            rb.start()
            ra.wait()
            accA()
            rb.wait()
            accB()

        g_ref[pl.ds(lo_a, 64), COLS[0]] = (
            out_ref[pl.ds(lo_a, 64), COLS[0]].astype(jnp.bfloat16)
        )
        g_ref[pl.ds(lo_b, 64), COLS[1]] = (
            out_ref[pl.ds(lo_b, 64), COLS[1]].astype(jnp.bfloat16)
        )

        ag_a = tuple(reversed(SCHED_A))
        ag_b = tuple(reversed(SCHED_B))
        cur_a, cur_b = lo_a, lo_b
        for j in range(4):
            sz = AG_SIZE[j]
            rdma_ga = pltpu.make_async_remote_copy(
                src_ref=g_ref.at[pl.ds(cur_a, sz), COLS[0]],
                dst_ref=g_ref.at[pl.ds(cur_a, sz), COLS[0]],
                send_sem=ag_send.at[j],
                recv_sem=ag_send.at[4 + j],
                device_id=(my ^ ag_a[j][0],),
                device_id_type=pl.DeviceIdType.MESH,
            )
            rdma_gb = pltpu.make_async_remote_copy(
                src_ref=g_ref.at[pl.ds(cur_b, sz), COLS[1]],
                dst_ref=g_ref.at[pl.ds(cur_b, sz), COLS[1]],
                send_sem=ag_recv.at[j],
                recv_sem=ag_recv.at[4 + j],
                device_id=(my ^ ag_b[j][0],),
                device_id_type=pl.DeviceIdType.MESH,
            )
            rdma_ga.start()
            rdma_gb.start()
            rdma_ga.wait()
            rdma_gb.wait()
            blk = 2 * sz
            cur_a = (cur_a // blk) * blk
            cur_b = (cur_b // blk) * blk

        out_ref[...] = g_ref[...].astype(jnp.float32)

    out = pl.pallas_call(
        body,
        out_shape=jax.ShapeDtypeStruct((SQ, 1024), jnp.float32),
        in_specs=[pl.BlockSpec(memory_space=pltpu.VMEM)] * 5,
        out_specs=pl.BlockSpec(memory_space=pltpu.VMEM),
        scratch_shapes=[
            pltpu.VMEM((SQ, 1024), jnp.bfloat16),
            pltpu.VMEM((960, 1024), jnp.bfloat16),
            pltpu.VMEM((960, 1024), jnp.bfloat16),
            pltpu.VMEM((SQ, 1024), jnp.bfloat16),
            pltpu.VMEM((SQ, 1024), jnp.bfloat16),
            pltpu.VMEM((SQ, SKV), jnp.float32),
            pltpu.SemaphoreType.DMA((8,)),
            pltpu.SemaphoreType.DMA((8,)),
            pltpu.SemaphoreType.DMA((8,)),
            pltpu.SemaphoreType.DMA((8,)),
        ],
        compiler_params=pltpu.CompilerParams(collective_id=0),
    )(x2, Wq_i, K2, V2, Wo_i)
    return out.reshape(1, SQ, 1024)


# device time: 79054 ns/iter; 1.0027x vs baseline; 1.0027x over previous
import jax
import jax.numpy as jnp
from jax import lax
from jax.experimental import pallas as pl
from jax.experimental.pallas import tpu as pltpu

N_DEV = 16
SQ = 1024
SKV = 1024
H_PER = 8
DH = 128
D_MODEL = 1024
HALF = SQ // 2
CHALF = 512
SCALE = 0.08838834764831843

SCHED_A = ((3, 2), (1, 1), (4, 4), (8, 8))
SCHED_B = ((4, 4), (8, 8), (3, 2), (1, 1))
RS_HALF = (512, 256, 128, 64)
RS_OFF = (0, 512, 768, 896)
AG_SIZE = (64, 128, 256, 512)


def kernel(x, Wq, K_ext, V_ext, Wo):
    d = lax.axis_index("i")

    x2 = x.reshape(SQ, 1024).astype(jnp.bfloat16)
    Wq_i = lax.dynamic_slice(Wq, (0, d * D_MODEL), (1024, D_MODEL)).astype(
        jnp.bfloat16
    )
    Wo_i = lax.dynamic_slice(Wo, (d * D_MODEL, 0), (D_MODEL, 1024)).astype(
        jnp.bfloat16
    )
    K2 = K_ext.reshape(SKV, H_PER * DH).astype(jnp.bfloat16)
    V2 = V_ext.reshape(SKV, H_PER * DH).astype(jnp.bfloat16)

    def body(x_ref, wq_ref, k_ref, v_ref, wo_ref, out_ref, g_ref,
             stage_ref, rsbuf_ref, q_scr, ctx_scr, bias_scr,
             rs_send, rs_recv, ag_send, ag_recv):
        my = lax.axis_index("i")
        COLS = (pl.ds(0, CHALF), pl.ds(CHALF, CHALF))

        q_scr[...] = (
            jnp.dot(x_ref[...], wq_ref[...],
                    preferred_element_type=jnp.float32) * SCALE
        ).astype(jnp.bfloat16)

        qi = lax.broadcasted_iota(jnp.int32, (SQ, SKV), 0)
        ki = lax.broadcasted_iota(jnp.int32, (SQ, SKV), 1)
        mask = (jnp.abs(qi - ki) <= 128) | (ki < 32) | (qi < 32)
        bias_scr[...] = jnp.where(mask, 0.0, -1e9)

        barrier_sem = pltpu.get_barrier_semaphore()
        for m_, _ in SCHED_A:
            pl.semaphore_signal(
                barrier_sem, inc=1,
                device_id=(my ^ m_,), device_id_type=pl.DeviceIdType.MESH,
            )
        pl.semaphore_wait(barrier_sem, 4)

        def compute_half(ro):
            boff = jnp.where(ro == 0, 128, 384).astype(jnp.int32)
            rows = pl.ds(ro, HALF)

            def head_body(h, carry):
                col = pl.ds(h * DH, DH)
                q_h = q_scr[rows, col]
                dims = (((1,), (1,)), ((), ()))
                s1 = lax.dot_general(
                    q_h, k_ref[pl.ds(0, 128), col],
                    dims, preferred_element_type=jnp.float32,
                ) + bias_scr[rows, pl.ds(0, 128)]
                s2 = lax.dot_general(
                    q_h, k_ref[pl.ds(boff, 640), col],
                    dims, preferred_element_type=jnp.float32,
                ) + bias_scr[rows, pl.ds(boff, 640)]
                e1 = jnp.exp(s1)
                e2 = jnp.exp(s2)
                den = (jnp.sum(e1, axis=1, keepdims=True)
                       + jnp.sum(e2, axis=1, keepdims=True))
                ctx = (
                    jnp.dot(e1.astype(jnp.bfloat16), v_ref[pl.ds(0, 128), col],
                            preferred_element_type=jnp.float32)
                    + jnp.dot(e2.astype(jnp.bfloat16),
                              v_ref[pl.ds(boff, 640), col],
                              preferred_element_type=jnp.float32)
                )
                q32 = q_scr[pl.ds(ro, 32), col]
                s3 = lax.dot_general(
                    q32, k_ref[pl.ds(768, 256), col],
                    dims, preferred_element_type=jnp.float32,
                ) + bias_scr[pl.ds(ro, 32), pl.ds(768, 256)]
                e3 = jnp.exp(s3)
                den3 = jnp.sum(e3, axis=1, keepdims=True)
                ctx3 = jnp.dot(
                    e3.astype(jnp.bfloat16), v_ref[pl.ds(768, 256), col],
                    preferred_element_type=jnp.float32,
                )
                ctx_scr[pl.ds(ro + 32, HALF - 32), col] = (
                    ctx[32:, :] * (1.0 / den[32:, :])
                ).astype(jnp.bfloat16)
                ctx_scr[pl.ds(ro, 32), col] = (
                    (ctx[:32, :] + ctx3) * (1.0 / (den[:32, :] + den3))
                ).astype(jnp.bfloat16)
                return carry

            lax.fori_loop(0, H_PER, head_body, 0)
            out_ref[rows, :] = jnp.dot(
                ctx_scr[rows, :], wo_ref[...],
                preferred_element_type=jnp.float32,
            )

        def rs_parts(k, lo, sched, side, sem_idx):
            m_, key = sched[k]
            half = RS_HALF[k]
            off = RS_OFF[k]
            hb = (my & key) != 0
            keep_lo = lo + jnp.where(hb, half, 0)
            send_lo = lo + jnp.where(hb, 0, half)
            cs = COLS[side]

            def stage():
                stage_ref[pl.ds(off, half), cs] = (
                    out_ref[pl.ds(send_lo, half), cs].astype(jnp.bfloat16)
                )

            rdma = pltpu.make_async_remote_copy(
                src_ref=stage_ref.at[pl.ds(off, half), cs],
                dst_ref=rsbuf_ref.at[pl.ds(off, half), cs],
                send_sem=rs_send.at[sem_idx],
                recv_sem=rs_recv.at[sem_idx],
                device_id=(my ^ m_,),
                device_id_type=pl.DeviceIdType.MESH,
            )

            def acc():
                rows = pl.ds(keep_lo, half)
                out_ref[rows, cs] = (
                    out_ref[rows, cs]
                    + rsbuf_ref[pl.ds(off, half), cs].astype(jnp.float32)
                )

            return stage, rdma, acc, keep_lo

        bitA = (my & SCHED_A[0][1]) != 0
        bitB = (my & SCHED_B[0][1]) != 0
        sendA = jnp.where(bitA, 0, HALF).astype(jnp.int32)
        same = bitA == bitB

        stA, rdma_a0, accA, lo_a = rs_parts(0, jnp.int32(0), SCHED_A, 0, 0)
        stB, rdma_b0, accB, lo_b = rs_parts(0, jnp.int32(0), SCHED_B, 1, 4)

        compute_half(sendA)
        stA()
        rdma_a0.start()

        @pl.when(same)
        def _():
            stB()
            rdma_b0.start()

        compute_half(HALF - sendA)

        @pl.when(jnp.logical_not(same))
        def _():
            stB()
            rdma_b0.start()

        rdma_a0.wait()
        accA()
        rdma_b0.wait()
        accB()

        for k in (1, 2, 3):
            stA, ra, accA, lo_a = rs_parts(k, lo_a, SCHED_A, 0, k)
            stB, rb, accB, lo_b = rs_parts(k, lo_b, SCHED_B, 1, 4 + k)
            stA()
            ra.start()
            stB()
            rb.start()
            ra.wait()
            accA()
            rb.wait()
            accB()

        g_ref[pl.ds(lo_a, 64), COLS[0]] = (
            out_ref[pl.ds(lo_a, 64), COLS[0]].astype(jnp.bfloat16)
        )
        g_ref[pl.ds(lo_b, 64), COLS[1]] = (
            out_ref[pl.ds(lo_b, 64), COLS[1]].astype(jnp.bfloat16)
        )

        ag_a = tuple(reversed(SCHED_A))
        ag_b = tuple(reversed(SCHED_B))
        cur_a, cur_b = lo_a, lo_b
        for j in range(4):
            sz = AG_SIZE[j]
            rdma_ga = pltpu.make_async_remote_copy(
                src_ref=g_ref.at[pl.ds(cur_a, sz), COLS[0]],
                dst_ref=g_ref.at[pl.ds(cur_a, sz), COLS[0]],
                send_sem=ag_send.at[j],
                recv_sem=ag_send.at[4 + j],
                device_id=(my ^ ag_a[j][0],),
                device_id_type=pl.DeviceIdType.MESH,
            )
            rdma_gb = pltpu.make_async_remote_copy(
                src_ref=g_ref.at[pl.ds(cur_b, sz), COLS[1]],
                dst_ref=g_ref.at[pl.ds(cur_b, sz), COLS[1]],
                send_sem=ag_recv.at[j],
                recv_sem=ag_recv.at[4 + j],
                device_id=(my ^ ag_b[j][0],),
                device_id_type=pl.DeviceIdType.MESH,
            )
            rdma_ga.start()
            rdma_gb.start()
            if j == 3:
                out_ref[pl.ds(cur_a, 512), COLS[0]] = (
                    g_ref[pl.ds(cur_a, 512), COLS[0]].astype(jnp.float32)
                )
                out_ref[pl.ds(cur_b, 512), COLS[1]] = (
                    g_ref[pl.ds(cur_b, 512), COLS[1]].astype(jnp.float32)
                )
                recv_a = 512 - cur_a
                recv_b = 512 - cur_b
            rdma_ga.wait()
            rdma_gb.wait()
            blk = 2 * sz
            cur_a = (cur_a // blk) * blk
            cur_b = (cur_b // blk) * blk

        out_ref[pl.ds(recv_a, 512), COLS[0]] = (
            g_ref[pl.ds(recv_a, 512), COLS[0]].astype(jnp.float32)
        )
        out_ref[pl.ds(recv_b, 512), COLS[1]] = (
            g_ref[pl.ds(recv_b, 512), COLS[1]].astype(jnp.float32)
        )

    out = pl.pallas_call(
        body,
        out_shape=jax.ShapeDtypeStruct((SQ, 1024), jnp.float32),
        in_specs=[pl.BlockSpec(memory_space=pltpu.VMEM)] * 5,
        out_specs=pl.BlockSpec(memory_space=pltpu.VMEM),
        scratch_shapes=[
            pltpu.VMEM((SQ, 1024), jnp.bfloat16),
            pltpu.VMEM((960, 1024), jnp.bfloat16),
            pltpu.VMEM((960, 1024), jnp.bfloat16),
            pltpu.VMEM((SQ, 1024), jnp.bfloat16),
            pltpu.VMEM((SQ, 1024), jnp.bfloat16),
            pltpu.VMEM((SQ, SKV), jnp.float32),
            pltpu.SemaphoreType.DMA((8,)),
            pltpu.SemaphoreType.DMA((8,)),
            pltpu.SemaphoreType.DMA((8,)),
            pltpu.SemaphoreType.DMA((8,)),
        ],
        compiler_params=pltpu.CompilerParams(collective_id=0),
    )(x2, Wq_i, K2, V2, Wo_i)
    return out.reshape(1, SQ, 1024)


# device time: 75181 ns/iter; 1.0543x vs baseline; 1.0515x over previous
import jax
import jax.numpy as jnp
from jax import lax
from jax.experimental import pallas as pl
from jax.experimental.pallas import tpu as pltpu

N_DEV = 16
SQ = 1024
SKV = 1024
H_PER = 8
DH = 128
D_MODEL = 1024
HALF = SQ // 2
CHALF = 512
SCALE = 0.08838834764831843

SCHED_A = ((3, 2), (1, 1), (4, 4), (8, 8))
SCHED_B = ((4, 4), (8, 8), (3, 2), (1, 1))
RS_HALF = (512, 256, 128, 64)
RS_OFF = (0, 512, 768, 896)
AG_SIZE = (64, 128, 256, 512)


def kernel(x, Wq, K_ext, V_ext, Wo):
    d = lax.axis_index("i")

    x2 = x.reshape(SQ, 1024)
    K2 = K_ext.reshape(SKV, H_PER * DH)
    V2 = V_ext.reshape(SKV, H_PER * DH)

    def body(x_ref, wq_hbm, k_raw, v_raw, wo_hbm, out_ref, g_ref,
             stage_ref, rsbuf_ref, q_scr, ctx_scr, bias_scr, k_ref, v_ref,
             wq_vmem, wo_vmem, w_sems,
             rs_send, rs_recv, ag_send, ag_recv):
        my = lax.axis_index("i")
        COLS = (pl.ds(0, CHALF), pl.ds(CHALF, CHALF))

        cp_wq = pltpu.make_async_copy(
            wq_hbm.at[:, pl.ds(my * D_MODEL, D_MODEL)], wq_vmem, w_sems.at[0]
        )
        cp_wo = pltpu.make_async_copy(
            wo_hbm.at[pl.ds(my * D_MODEL, D_MODEL), :], wo_vmem, w_sems.at[1]
        )
        cp_wq.start()
        cp_wo.start()

        k_ref[...] = k_raw[...].astype(jnp.bfloat16)
        v_ref[...] = v_raw[...].astype(jnp.bfloat16)

        qi = lax.broadcasted_iota(jnp.int32, (SQ, SKV), 0)
        ki = lax.broadcasted_iota(jnp.int32, (SQ, SKV), 1)
        mask = (jnp.abs(qi - ki) <= 128) | (ki < 32) | (qi < 32)
        bias_scr[...] = jnp.where(mask, 0.0, -1e9)

        cp_wq.wait()
        q_scr[...] = (
            jnp.dot(x_ref[...].astype(jnp.bfloat16),
                    wq_vmem[...].astype(jnp.bfloat16),
                    preferred_element_type=jnp.float32) * SCALE
        ).astype(jnp.bfloat16)
        cp_wo.wait()

        barrier_sem = pltpu.get_barrier_semaphore()
        for m_, _ in SCHED_A:
            pl.semaphore_signal(
                barrier_sem, inc=1,
                device_id=(my ^ m_,), device_id_type=pl.DeviceIdType.MESH,
            )
        pl.semaphore_wait(barrier_sem, 4)

        def compute_half(ro):
            boff = jnp.where(ro == 0, 128, 384).astype(jnp.int32)
            rows = pl.ds(ro, HALF)

            def head_body(h, carry):
                col = pl.ds(h * DH, DH)
                q_h = q_scr[rows, col]
                dims = (((1,), (1,)), ((), ()))
                s1 = lax.dot_general(
                    q_h, k_ref[pl.ds(0, 128), col],
                    dims, preferred_element_type=jnp.float32,
                ) + bias_scr[rows, pl.ds(0, 128)]
                s2 = lax.dot_general(
                    q_h, k_ref[pl.ds(boff, 640), col],
                    dims, preferred_element_type=jnp.float32,
                ) + bias_scr[rows, pl.ds(boff, 640)]
                e1 = jnp.exp(s1)
                e2 = jnp.exp(s2)
                den = (jnp.sum(e1, axis=1, keepdims=True)
                       + jnp.sum(e2, axis=1, keepdims=True))
                ctx = (
                    jnp.dot(e1.astype(jnp.bfloat16), v_ref[pl.ds(0, 128), col],
                            preferred_element_type=jnp.float32)
                    + jnp.dot(e2.astype(jnp.bfloat16),
                              v_ref[pl.ds(boff, 640), col],
                              preferred_element_type=jnp.float32)
                )
                q32 = q_scr[pl.ds(ro, 32), col]
                s3 = lax.dot_general(
                    q32, k_ref[pl.ds(768, 256), col],
                    dims, preferred_element_type=jnp.float32,
                ) + bias_scr[pl.ds(ro, 32), pl.ds(768, 256)]
                e3 = jnp.exp(s3)
                den3 = jnp.sum(e3, axis=1, keepdims=True)
                ctx3 = jnp.dot(
                    e3.astype(jnp.bfloat16), v_ref[pl.ds(768, 256), col],
                    preferred_element_type=jnp.float32,
                )
                ctx_scr[pl.ds(ro + 32, HALF - 32), col] = (
                    ctx[32:, :] * (1.0 / den[32:, :])
                ).astype(jnp.bfloat16)
                ctx_scr[pl.ds(ro, 32), col] = (
                    (ctx[:32, :] + ctx3) * (1.0 / (den[:32, :] + den3))
                ).astype(jnp.bfloat16)
                return carry

            lax.fori_loop(0, H_PER, head_body, 0)
            out_ref[rows, :] = jnp.dot(
                ctx_scr[rows, :], wo_vmem[...].astype(jnp.bfloat16),
                preferred_element_type=jnp.float32,
            )

        def rs_parts(k, lo, sched, side, sem_idx):
            m_, key = sched[k]
            half = RS_HALF[k]
            off = RS_OFF[k]
            hb = (my & key) != 0
            keep_lo = lo + jnp.where(hb, half, 0)
            send_lo = lo + jnp.where(hb, 0, half)
            cs = COLS[side]

            def stage():
                stage_ref[pl.ds(off, half), cs] = (
                    out_ref[pl.ds(send_lo, half), cs].astype(jnp.bfloat16)
                )

            rdma = pltpu.make_async_remote_copy(
                src_ref=stage_ref.at[pl.ds(off, half), cs],
                dst_ref=rsbuf_ref.at[pl.ds(off, half), cs],
                send_sem=rs_send.at[sem_idx],
                recv_sem=rs_recv.at[sem_idx],
                device_id=(my ^ m_,),
                device_id_type=pl.DeviceIdType.MESH,
            )

            def acc():
                rows = pl.ds(keep_lo, half)
                out_ref[rows, cs] = (
                    out_ref[rows, cs]
                    + rsbuf_ref[pl.ds(off, half), cs].astype(jnp.float32)
                )

            return stage, rdma, acc, keep_lo

        bitA = (my & SCHED_A[0][1]) != 0
        bitB = (my & SCHED_B[0][1]) != 0
        sendA = jnp.where(bitA, 0, HALF).astype(jnp.int32)
        same = bitA == bitB

        stA, rdma_a0, accA, lo_a = rs_parts(0, jnp.int32(0), SCHED_A, 0, 0)
        stB, rdma_b0, accB, lo_b = rs_parts(0, jnp.int32(0), SCHED_B, 1, 4)

        compute_half(sendA)
        stA()
        rdma_a0.start()

        @pl.when(same)
        def _():
            stB()
            rdma_b0.start()

        compute_half(HALF - sendA)

        @pl.when(jnp.logical_not(same))
        def _():
            stB()
            rdma_b0.start()

        rdma_a0.wait()
        accA()
        rdma_b0.wait()
        accB()

        for k in (1, 2, 3):
            stA, ra, accA, lo_a = rs_parts(k, lo_a, SCHED_A, 0, k)
            stB, rb, accB, lo_b = rs_parts(k, lo_b, SCHED_B, 1, 4 + k)
            stA()
            ra.start()
            stB()
            rb.start()
            ra.wait()
            accA()
            rb.wait()
            accB()

        g_ref[pl.ds(lo_a, 64), COLS[0]] = (
            out_ref[pl.ds(lo_a, 64), COLS[0]].astype(jnp.bfloat16)
        )
        g_ref[pl.ds(lo_b, 64), COLS[1]] = (
            out_ref[pl.ds(lo_b, 64), COLS[1]].astype(jnp.bfloat16)
        )

        ag_a = tuple(reversed(SCHED_A))
        ag_b = tuple(reversed(SCHED_B))
        cur_a, cur_b = lo_a, lo_b
        for j in range(4):
            sz = AG_SIZE[j]
            rdma_ga = pltpu.make_async_remote_copy(
                src_ref=g_ref.at[pl.ds(cur_a, sz), COLS[0]],
                dst_ref=g_ref.at[pl.ds(cur_a, sz), COLS[0]],
                send_sem=ag_send.at[j],
                recv_sem=ag_send.at[4 + j],
                device_id=(my ^ ag_a[j][0],),
                device_id_type=pl.DeviceIdType.MESH,
            )
            rdma_gb = pltpu.make_async_remote_copy(
                src_ref=g_ref.at[pl.ds(cur_b, sz), COLS[1]],
                dst_ref=g_ref.at[pl.ds(cur_b, sz), COLS[1]],
                send_sem=ag_recv.at[j],
                recv_sem=ag_recv.at[4 + j],
                device_id=(my ^ ag_b[j][0],),
                device_id_type=pl.DeviceIdType.MESH,
            )
            rdma_ga.start()
            rdma_gb.start()
            if j == 3:
                out_ref[pl.ds(cur_a, 512), COLS[0]] = (
                    g_ref[pl.ds(cur_a, 512), COLS[0]].astype(jnp.float32)
                )
                out_ref[pl.ds(cur_b, 512), COLS[1]] = (
                    g_ref[pl.ds(cur_b, 512), COLS[1]].astype(jnp.float32)
                )
                recv_a = 512 - cur_a
                recv_b = 512 - cur_b
            rdma_ga.wait()
            rdma_gb.wait()
            blk = 2 * sz
            cur_a = (cur_a // blk) * blk
            cur_b = (cur_b // blk) * blk

        out_ref[pl.ds(recv_a, 512), COLS[0]] = (
            g_ref[pl.ds(recv_a, 512), COLS[0]].astype(jnp.float32)
        )
        out_ref[pl.ds(recv_b, 512), COLS[1]] = (
            g_ref[pl.ds(recv_b, 512), COLS[1]].astype(jnp.float32)
        )

    out = pl.pallas_call(
        body,
        out_shape=jax.ShapeDtypeStruct((SQ, 1024), jnp.float32),
        in_specs=[
            pl.BlockSpec(memory_space=pltpu.VMEM),
            pl.BlockSpec(memory_space=pltpu.MemorySpace.HBM),
            pl.BlockSpec(memory_space=pltpu.VMEM),
            pl.BlockSpec(memory_space=pltpu.VMEM),
            pl.BlockSpec(memory_space=pltpu.MemorySpace.HBM),
        ],
        out_specs=pl.BlockSpec(memory_space=pltpu.VMEM),
        scratch_shapes=[
            pltpu.VMEM((SQ, 1024), jnp.bfloat16),
            pltpu.VMEM((960, 1024), jnp.bfloat16),
            pltpu.VMEM((960, 1024), jnp.bfloat16),
            pltpu.VMEM((SQ, 1024), jnp.bfloat16),
            pltpu.VMEM((SQ, 1024), jnp.bfloat16),
            pltpu.VMEM((SQ, SKV), jnp.float32),
            pltpu.VMEM((SKV, 1024), jnp.bfloat16),
            pltpu.VMEM((SKV, 1024), jnp.bfloat16),
            pltpu.VMEM((1024, D_MODEL), jnp.float32),
            pltpu.VMEM((D_MODEL, 1024), jnp.float32),
            pltpu.SemaphoreType.DMA((2,)),
            pltpu.SemaphoreType.DMA((8,)),
            pltpu.SemaphoreType.DMA((8,)),
            pltpu.SemaphoreType.DMA((8,)),
            pltpu.SemaphoreType.DMA((8,)),
        ],
        compiler_params=pltpu.CompilerParams(collective_id=0),
    )(x2, Wq, K2, V2, Wo)
    return out.reshape(1, SQ, 1024)


# device time: 70272 ns/iter; 1.1280x vs baseline; 1.0699x over previous
import jax
import jax.numpy as jnp
from jax import lax
from jax.experimental import pallas as pl
from jax.experimental.pallas import tpu as pltpu

N_DEV = 16
SQ = 1024
SKV = 1024
H_PER = 8
DH = 128
D_MODEL = 1024
HALF = SQ // 2
CHALF = 512
SCALE = 0.08838834764831843

SCHED_A = ((3, 2), (1, 1), (4, 4), (8, 8))
SCHED_B = ((4, 4), (8, 8), (3, 2), (1, 1))
RS_HALF = (512, 256, 128, 64)
RS_OFF = (0, 512, 768, 896)
AG_SIZE = (64, 128, 256, 512)


def kernel(x, Wq, K_ext, V_ext, Wo):
    d = lax.axis_index("i")

    x2 = x.reshape(SQ, 1024)
    K2 = K_ext.reshape(SKV, H_PER * DH)
    V2 = V_ext.reshape(SKV, H_PER * DH)

    def body(x_ref, wq_hbm, k_raw, v_raw, wo_hbm, out_ref, g_ref,
             stage_ref, rsbuf_ref, q_scr, ctx_scr, bias_scr, k_ref, v_ref,
             wq_vmem, wo_vmem, w_sems,
             rs_send, rs_recv, ag_send, ag_recv):
        my = lax.axis_index("i")
        COLS = (pl.ds(0, CHALF), pl.ds(CHALF, CHALF))

        cp_wq = pltpu.make_async_copy(
            wq_hbm.at[:, pl.ds(my * D_MODEL, D_MODEL)], wq_vmem, w_sems.at[0]
        )
        cp_wo = pltpu.make_async_copy(
            wo_hbm.at[pl.ds(my * D_MODEL, D_MODEL), :], wo_vmem, w_sems.at[1]
        )
        cp_wq.start()
        cp_wo.start()

        k_ref[...] = k_raw[...].astype(jnp.bfloat16)
        v_ref[...] = v_raw[...].astype(jnp.bfloat16)

        qi = lax.broadcasted_iota(jnp.int32, (SQ, SKV), 0)
        ki = lax.broadcasted_iota(jnp.int32, (SQ, SKV), 1)
        mask = (jnp.abs(qi - ki) <= 128) | (ki < 32) | (qi < 32)
        bias_scr[...] = jnp.where(mask, 0.0, -1e9)

        cp_wq.wait()
        q_scr[...] = (
            jnp.dot(x_ref[...].astype(jnp.bfloat16),
                    wq_vmem[...].astype(jnp.bfloat16),
                    preferred_element_type=jnp.float32) * SCALE
        ).astype(jnp.bfloat16)
        cp_wo.wait()

        barrier_sem = pltpu.get_barrier_semaphore()
        for m_, _ in SCHED_A:
            pl.semaphore_signal(
                barrier_sem, inc=1,
                device_id=(my ^ m_,), device_id_type=pl.DeviceIdType.MESH,
            )
        pl.semaphore_wait(barrier_sem, 4)

        def compute_half(ro):
            boff = jnp.where(ro == 0, 128, 384).astype(jnp.int32)
            rows = pl.ds(ro, HALF)

            def head_body(h, carry):
                col = pl.ds(h * DH, DH)
                q_h = q_scr[rows, col]
                dims = (((1,), (1,)), ((), ()))
                s1 = lax.dot_general(
                    q_h, k_ref[pl.ds(0, 128), col],
                    dims, preferred_element_type=jnp.float32,
                ) + bias_scr[rows, pl.ds(0, 128)]
                s2 = lax.dot_general(
                    q_h, k_ref[pl.ds(boff, 640), col],
                    dims, preferred_element_type=jnp.float32,
                ) + bias_scr[rows, pl.ds(boff, 640)]
                e1 = jnp.exp(s1)
                e2 = jnp.exp(s2)
                den = (jnp.sum(e1, axis=1, keepdims=True)
                       + jnp.sum(e2, axis=1, keepdims=True))
                ctx = (
                    jnp.dot(e1.astype(jnp.bfloat16), v_ref[pl.ds(0, 128), col],
                            preferred_element_type=jnp.float32)
                    + jnp.dot(e2.astype(jnp.bfloat16),
                              v_ref[pl.ds(boff, 640), col],
                              preferred_element_type=jnp.float32)
                )
                q32 = q_scr[pl.ds(ro, 32), col]
                s3 = lax.dot_general(
                    q32, k_ref[pl.ds(768, 256), col],
                    dims, preferred_element_type=jnp.float32,
                ) + bias_scr[pl.ds(ro, 32), pl.ds(768, 256)]
                e3 = jnp.exp(s3)
                den3 = jnp.sum(e3, axis=1, keepdims=True)
                ctx3 = jnp.dot(
                    e3.astype(jnp.bfloat16), v_ref[pl.ds(768, 256), col],
                    preferred_element_type=jnp.float32,
                )
                ctx_scr[pl.ds(ro + 32, HALF - 32), col] = (
                    ctx[32:, :] * (1.0 / den[32:, :])
                ).astype(jnp.bfloat16)
                ctx_scr[pl.ds(ro, 32), col] = (
                    (ctx[:32, :] + ctx3) * (1.0 / (den[:32, :] + den3))
                ).astype(jnp.bfloat16)
                return carry

            lax.fori_loop(0, H_PER, head_body, 0)
            out_ref[rows, :] = jnp.dot(
                ctx_scr[rows, :], wo_vmem[...].astype(jnp.bfloat16),
                preferred_element_type=jnp.float32,
            )

        def rs_parts(k, lo, sched, side, sem_idx):
            m_, key = sched[k]
            half = RS_HALF[k]
            off = RS_OFF[k]
            hb = (my & key) != 0
            keep_lo = lo + jnp.where(hb, half, 0)
            send_lo = lo + jnp.where(hb, 0, half)
            cs = COLS[side]

            def stage():
                stage_ref[pl.ds(off, half), cs] = (
                    out_ref[pl.ds(send_lo, half), cs].astype(jnp.bfloat16)
                )

            rdma = pltpu.make_async_remote_copy(
                src_ref=stage_ref.at[pl.ds(off, half), cs],
                dst_ref=rsbuf_ref.at[pl.ds(off, half), cs],
                send_sem=rs_send.at[sem_idx],
                recv_sem=rs_recv.at[sem_idx],
                device_id=(my ^ m_,),
                device_id_type=pl.DeviceIdType.MESH,
            )

            def acc():
                rows = pl.ds(keep_lo, half)
                out_ref[rows, cs] = (
                    out_ref[rows, cs]
                    + rsbuf_ref[pl.ds(off, half), cs].astype(jnp.float32)
                )

            return stage, rdma, acc, keep_lo

        bitA = (my & SCHED_A[0][1]) != 0
        bitB = (my & SCHED_B[0][1]) != 0
        sendB = jnp.where(bitB, 0, HALF).astype(jnp.int32)
        same = bitA == bitB

        stA, rdma_a0, accA, lo_a = rs_parts(0, jnp.int32(0), SCHED_A, 0, 0)
        stB, rdma_b0, accB, lo_b = rs_parts(0, jnp.int32(0), SCHED_B, 1, 4)

        compute_half(sendB)
        stB()
        rdma_b0.start()

        @pl.when(same)
        def _():
            stA()
            rdma_a0.start()

        compute_half(HALF - sendB)

        @pl.when(jnp.logical_not(same))
        def _():
            stA()
            rdma_a0.start()

        rdma_b0.wait()
        accB()
        stB, rb, accB, lo_b = rs_parts(1, lo_b, SCHED_B, 1, 5)
        stB()
        rb.start()

        rdma_a0.wait()
        accA()
        stA, ra, accA, lo_a = rs_parts(1, lo_a, SCHED_A, 0, 1)
        stA()
        ra.start()

        for k in (2, 3):
            rb.wait()
            accB()
            stB, rb, accB, lo_b = rs_parts(k, lo_b, SCHED_B, 1, 4 + k)
            stB()
            rb.start()
            ra.wait()
            accA()
            stA, ra, accA, lo_a = rs_parts(k, lo_a, SCHED_A, 0, k)
            stA()
            ra.start()

        ag_a = tuple(reversed(SCHED_A))
        ag_b = tuple(reversed(SCHED_B))

        def make_ag(side, sched, cur, j):
            sems = ag_send if side == 0 else ag_recv
            return pltpu.make_async_remote_copy(
                src_ref=g_ref.at[pl.ds(cur, AG_SIZE[j]), COLS[side]],
                dst_ref=g_ref.at[pl.ds(cur, AG_SIZE[j]), COLS[side]],
                send_sem=sems.at[j],
                recv_sem=sems.at[4 + j],
                device_id=(my ^ sched[j][0],),
                device_id_type=pl.DeviceIdType.MESH,
            )

        rb.wait()
        accB()
        g_ref[pl.ds(lo_b, 64), COLS[1]] = (
            out_ref[pl.ds(lo_b, 64), COLS[1]].astype(jnp.bfloat16)
        )
        cur_b = lo_b
        gb = make_ag(1, ag_b, cur_b, 0)
        gb.start()

        ra.wait()
        accA()
        g_ref[pl.ds(lo_a, 64), COLS[0]] = (
            out_ref[pl.ds(lo_a, 64), COLS[0]].astype(jnp.bfloat16)
        )
        cur_a = lo_a
        ga = make_ag(0, ag_a, cur_a, 0)
        ga.start()

        for j in (1, 2, 3):
            gb.wait()
            cur_b = (cur_b // (2 * AG_SIZE[j - 1])) * (2 * AG_SIZE[j - 1])
            gb = make_ag(1, ag_b, cur_b, j)
            gb.start()
            ga.wait()
            cur_a = (cur_a // (2 * AG_SIZE[j - 1])) * (2 * AG_SIZE[j - 1])
            ga = make_ag(0, ag_a, cur_a, j)
            ga.start()
            if j == 3:
                out_ref[pl.ds(cur_b, 512), COLS[1]] = (
                    g_ref[pl.ds(cur_b, 512), COLS[1]].astype(jnp.float32)
                )
                out_ref[pl.ds(cur_a, 512), COLS[0]] = (
                    g_ref[pl.ds(cur_a, 512), COLS[0]].astype(jnp.float32)
                )
                recv_a = 512 - cur_a
                recv_b = 512 - cur_b
        gb.wait()
        ga.wait()

        out_ref[pl.ds(recv_a, 512), COLS[0]] = (
            g_ref[pl.ds(recv_a, 512), COLS[0]].astype(jnp.float32)
        )
        out_ref[pl.ds(recv_b, 512), COLS[1]] = (
            g_ref[pl.ds(recv_b, 512), COLS[1]].astype(jnp.float32)
        )

    out = pl.pallas_call(
        body,
        out_shape=jax.ShapeDtypeStruct((SQ, 1024), jnp.float32),
        in_specs=[
            pl.BlockSpec(memory_space=pltpu.VMEM),
            pl.BlockSpec(memory_space=pltpu.MemorySpace.HBM),
            pl.BlockSpec(memory_space=pltpu.VMEM),
            pl.BlockSpec(memory_space=pltpu.VMEM),
            pl.BlockSpec(memory_space=pltpu.MemorySpace.HBM),
        ],
        out_specs=pl.BlockSpec(memory_space=pltpu.VMEM),
        scratch_shapes=[
            pltpu.VMEM((SQ, 1024), jnp.bfloat16),
            pltpu.VMEM((960, 1024), jnp.bfloat16),
            pltpu.VMEM((960, 1024), jnp.bfloat16),
            pltpu.VMEM((SQ, 1024), jnp.bfloat16),
            pltpu.VMEM((SQ, 1024), jnp.bfloat16),
            pltpu.VMEM((SQ, SKV), jnp.float32),
            pltpu.VMEM((SKV, 1024), jnp.bfloat16),
            pltpu.VMEM((SKV, 1024), jnp.bfloat16),
            pltpu.VMEM((1024, D_MODEL), jnp.float32),
            pltpu.VMEM((D_MODEL, 1024), jnp.float32),
            pltpu.SemaphoreType.DMA((2,)),
            pltpu.SemaphoreType.DMA((8,)),
            pltpu.SemaphoreType.DMA((8,)),
            pltpu.SemaphoreType.DMA((8,)),
            pltpu.SemaphoreType.DMA((8,)),
        ],
        compiler_params=pltpu.CompilerParams(collective_id=0),
    )(x2, Wq, K2, V2, Wo)
    return out.reshape(1, SQ, 1024)
